# baseline (speedup 1.0000x reference)
import functools

import jax
import jax.numpy as jnp
from jax import lax
from jax.experimental import pallas as pl
from jax.experimental.pallas import tpu as pltpu
from jax.experimental.pallas import tpu_sc as plsc

MEMORY_SIZE = 100000
NGRAM_N = 4
NUM_HEADS = 4
HEAD_DIM = 128
EMBED_DIM = NUM_HEADS * HEAD_DIM
B, W, O = 1024, 50, 50
SEQ_LEN = O + W

_WRAP_MOD = (1 << 32) % MEMORY_SIZE


def _prime_table():
    ps = []
    base = 131
    for h in range(NUM_HEADS):
        x, r = base + h * 1009, []
        for _ in range(NGRAM_N):
            r.append(x)
            x = x * 31 + 1
        ps.append(r)
    return ps


_PRIMES = _prime_table()

NC, NS = 2, 16
NW = NC * NS
QTOT = B * W
QW = QTOT // NW
B_PER = B // NW
CQ = 32
RCH = CQ * NUM_HEADS
NCH = QW // CQ
NGB = 5
NOB = 2
PREF = 3


@functools.lru_cache(maxsize=None)
def _build_engram_sc():
    mesh = plsc.VectorSubcoreMesh(core_axis_name="c", subcore_axis_name="s")
    return functools.partial(
        pl.kernel,
        mesh=mesh,
        out_type=jax.ShapeDtypeStruct((NUM_HEADS * QTOT, HEAD_DIM), jnp.float32),
        scratch_types=[
            pltpu.VMEM((B_PER, SEQ_LEN), jnp.int32),
            pltpu.VMEM((NCH, RCH), jnp.int32),
        ]
        + [pltpu.VMEM((RCH, HEAD_DIM), jnp.float32) for _ in range(NGB)]
        + [pltpu.VMEM((RCH, HEAD_DIM), jnp.float32) for _ in range(NOB)]
        + [pltpu.SemaphoreType.DMA for _ in range(NGB + NOB)],
        compiler_params=pltpu.CompilerParams(needs_layout_passes=False),
    )(_engram_sc)


def _engram_sc(seq_hbm, table_hbm, out_hbm, seq_v, idx_v, *bufs_sems):
    gbufs = bufs_sems[:NGB]
    obufs = bufs_sems[NGB : NGB + NOB]
    gsems = bufs_sems[NGB + NOB : 2 * NGB + NOB]
    wsems = bufs_sems[2 * NGB + NOB :]

    wid = lax.axis_index("s") * NC + lax.axis_index("c")
    b0 = wid * B_PER
    wq0 = wid * QW

    pltpu.sync_copy(seq_hbm.at[pl.ds(b0, B_PER), :], seq_v)

    lanes = lax.iota(jnp.int32, 16)

    def hash_body(k, carry):
        qv = wq0 + k * 16 + lanes
        b = lax.div(qv, jnp.int32(W))
        w = qv - b * W
        brel = b - b0
        vals = []
        for i in range(NGRAM_N):
            col = w + (O - i)
            vals.append(plsc.load_gather(seq_v, [brel, col]))
        pos0 = (k * 16 + lanes) * NUM_HEADS
        for h in range(NUM_HEADS):
            hs = vals[0] * jnp.int32(_PRIMES[0][h])
            for i in range(1, NGRAM_N):
                hs = hs + vals[i] * jnp.int32(_PRIMES[i][h])
            m = lax.rem(hs, jnp.int32(MEMORY_SIZE))
            m = jnp.where(m < 0, m + MEMORY_SIZE, m)
            m = jnp.where(hs < 0, m + _WRAP_MOD, m)
            m = jnp.where(m >= MEMORY_SIZE, m - MEMORY_SIZE, m)
            fidx = m * NUM_HEADS + h
            pos = pos0 + h
            plsc.store_scatter(idx_v, [pos >> 7, pos & 127], fidx)
        return carry

    lax.fori_loop(0, QW // 16, hash_body, 0)

    def fire_gather(c, j):
        pltpu.async_copy(table_hbm.at[idx_v.at[c]], gbufs[j], gsems[j])

    def wait_gather(j):
        pltpu.make_async_copy(
            table_hbm.at[pl.ds(0, RCH), :], gbufs[j], gsems[j]
        ).wait()

    def fire_write(c, j):
        qb = wq0 + c * CQ
        for h in range(NUM_HEADS):
            pltpu.async_copy(
                obufs[j].at[pl.ds(h * CQ, CQ), :],
                out_hbm.at[pl.ds(h * QTOT + qb, CQ), :],
                wsems[j],
            )

    def wait_write(j):
        pltpu.make_async_copy(
            obufs[j], out_hbm.at[pl.ds(wq0, RCH), :], wsems[j]
        ).wait()

    def repack(jg, jo):
        src = gbufs[jg]
        dst = obufs[jo]

        def rp_body(u, carry2):
            r = u * NUM_HEADS
            for h in range(NUM_HEADS):
                for v in range(HEAD_DIM // 16):
                    sl = pl.ds(v * 16, 16)
                    dst[h * CQ + u, sl] = src[r + h, sl]
            return carry2

        lax.fori_loop(0, CQ, rp_body, 0)

    for j in range(PREF):
        fire_gather(j, j)

    PER = 10

    def pipe_body(t, carry):
        for jj in range(PER):
            c = t * PER + jj
            jg = jj % NGB
            jo = jj % NOB
            wait_gather(jg)

            @pl.when(c >= NOB)
            def _():
                wait_write(jo)

            repack(jg, jo)
            fire_write(c, jo)

            @pl.when(c + PREF < NCH)
            def _():
                fire_gather(c + PREF, (jj + PREF) % NGB)
        return carry

    lax.fori_loop(0, NCH // PER, pipe_body, 0)

    for j in range(NOB):
        wait_write(j)


def _finish_body(g_ref, x0_ref, x1_ref, x2_ref, x3_ref, o_ref):
    g = jax.nn.sigmoid(g_ref[...])
    for h, xr in enumerate((x0_ref, x1_ref, x2_ref, x3_ref)):
        gh = g[0, h * HEAD_DIM : (h + 1) * HEAD_DIM]
        o_ref[:, :, h * HEAD_DIM : (h + 1) * HEAD_DIM] = (
            xr[...].reshape(8, W, HEAD_DIM) * gh[None, None, :]
        )


@functools.lru_cache(maxsize=None)
def _build_finish_tc():
    def head_spec(h):
        return pl.BlockSpec((8 * W, HEAD_DIM), lambda i, _h=h: (_h * (B // 8) + i, 0))

    return pl.pallas_call(
        _finish_body,
        grid=(B // 8,),
        in_specs=[pl.BlockSpec((1, EMBED_DIM), lambda i: (0, 0))]
        + [head_spec(h) for h in range(NUM_HEADS)],
        out_specs=pl.BlockSpec((8, W, EMBED_DIM), lambda i: (i, 0, 0)),
        out_shape=jax.ShapeDtypeStruct((B, W, EMBED_DIM), jnp.float32),
    )


@jax.jit
def kernel(curr, prev, table, gate):
    seq = jnp.concatenate([prev, curr], axis=1)
    table_flat = table.reshape(MEMORY_SIZE * NUM_HEADS, HEAD_DIM)
    inter = _build_engram_sc()(seq, table_flat)
    g2 = gate.reshape(1, EMBED_DIM)
    return _build_finish_tc()(g2, inter, inter, inter, inter)


# speedup vs baseline: 2.1676x; 2.1676x over previous
import functools

import jax
import jax.numpy as jnp
from jax import lax
from jax.experimental import pallas as pl
from jax.experimental.pallas import tpu as pltpu
from jax.experimental.pallas import tpu_sc as plsc

MEMORY_SIZE = 100000
NGRAM_N = 4
NUM_HEADS = 4
HEAD_DIM = 128
EMBED_DIM = NUM_HEADS * HEAD_DIM
B, W, O = 1024, 50, 50
SEQ_LEN = O + W

_WRAP_MOD = (1 << 32) % MEMORY_SIZE


def _prime_table():
    ps = []
    base = 131
    for h in range(NUM_HEADS):
        x, r = base + h * 1009, []
        for _ in range(NGRAM_N):
            r.append(x)
            x = x * 31 + 1
        ps.append(r)
    return ps


_PRIMES = _prime_table()

NC, NS = 2, 16
NW = NC * NS
TB = B // 8
NPAIR = W * TB
PW = NPAIR // NW
CP = 4
RCH = CP * 8 * NUM_HEADS
NCH = PW // CP
SROWS = 16
SEQ_PAD = 104


@functools.lru_cache(maxsize=None)
def _build_engram_sc():
    mesh = plsc.VectorSubcoreMesh(core_axis_name="c", subcore_axis_name="s")
    return functools.partial(
        pl.kernel,
        mesh=mesh,
        out_type=jax.ShapeDtypeStruct((B, W, EMBED_DIM), jnp.float32),
        scratch_types=[
            pltpu.VMEM((SROWS, B), jnp.int32),
            pltpu.VMEM((NUM_HEADS, HEAD_DIM), jnp.float32),
            pltpu.VMEM((NCH, RCH), jnp.int32),
            pltpu.VMEM((RCH, HEAD_DIM), jnp.float32),
            pltpu.VMEM((RCH, HEAD_DIM), jnp.float32),
            pltpu.VMEM((CP * 8, 1, EMBED_DIM), jnp.float32),
            pltpu.VMEM((CP * 8, 1, EMBED_DIM), jnp.float32),
            pltpu.SemaphoreType.DMA,
            pltpu.SemaphoreType.DMA,
            pltpu.SemaphoreType.DMA,
            pltpu.SemaphoreType.DMA,
        ],
        compiler_params=pltpu.CompilerParams(needs_layout_passes=False),
    )(_engram_sc)


def _engram_sc(
    seqt_hbm,
    table_hbm,
    gate_hbm,
    out_hbm,
    seqt_v,
    g_v,
    idx_v,
    ga0,
    ga1,
    ob0,
    ob1,
    gsem0,
    gsem1,
    wsem0,
    wsem1,
):
    gbufs = (ga0, ga1)
    obufs = (ob0, ob1)
    gsems = (gsem0, gsem1)
    wsems = (wsem0, wsem1)

    wid = lax.axis_index("s") * NC + lax.axis_index("c")
    wp0 = wid * PW
    wmin = lax.div(wp0, jnp.int32(TB))
    sbase = lax.div(wmin + (O - NGRAM_N + 1), jnp.int32(8)) * 8
    sbase = pl.multiple_of(sbase, 8)

    pltpu.sync_copy(seqt_hbm.at[pl.ds(sbase, SROWS), :], seqt_v)
    pltpu.sync_copy(gate_hbm, g_v)
    for h in range(NUM_HEADS):
        for v in range(HEAD_DIM // 16):
            sl = pl.ds(v * 16, 16)
            x = g_v[h, sl]
            g_v[h, sl] = 1.0 / (1.0 + jnp.exp(-x))

    lanes = lax.iota(jnp.int32, 16)

    def hash_body(k, carry):
        pl_loc = 2 * k + (lanes >> 3)
        p = wp0 + pl_loc
        w = lax.div(p, jnp.int32(TB))
        tb = p - w * TB
        b = tb * 8 + (lanes & 7)
        s = pl_loc * 8 + (lanes & 7)
        vals = []
        for i in range(NGRAM_N):
            col = w + (O - i) - sbase
            vals.append(plsc.load_gather(seqt_v, [col, b]))
        pos0 = s * NUM_HEADS
        for h in range(NUM_HEADS):
            hs = vals[0] * jnp.int32(_PRIMES[0][h])
            for i in range(1, NGRAM_N):
                hs = hs + vals[i] * jnp.int32(_PRIMES[i][h])
            m = lax.rem(hs, jnp.int32(MEMORY_SIZE))
            m = jnp.where(m < 0, m + MEMORY_SIZE, m)
            m = jnp.where(hs < 0, m + _WRAP_MOD, m)
            m = jnp.where(m >= MEMORY_SIZE, m - MEMORY_SIZE, m)
            fidx = m * NUM_HEADS + h
            pos = pos0 + h
            plsc.store_scatter(idx_v, [pos >> 7, pos & 127], fidx)
        return carry

    lax.fori_loop(0, PW * 8 // 16, hash_body, 0)

    def fire_gather(c, j):
        pltpu.async_copy(table_hbm.at[idx_v.at[c]], gbufs[j], gsems[j])

    def wait_gather(j):
        pltpu.make_async_copy(
            table_hbm.at[pl.ds(0, RCH), :], gbufs[j], gsems[j]
        ).wait()

    def fire_write(c, j):
        pg = wp0 + c * CP
        wc = lax.div(pg, jnp.int32(TB))
        tbase = (pg - wc * TB) * 8
        pltpu.async_copy(
            obufs[j],
            out_hbm.at[pl.ds(tbase, CP * 8), pl.ds(wc, 1), :],
            wsems[j],
        )

    def wait_write(j):
        pltpu.make_async_copy(
            obufs[j], out_hbm.at[pl.ds(0, CP * 8), pl.ds(0, 1), :], wsems[j]
        ).wait()

    gv = [
        [g_v[h, pl.ds(v * 16, 16)] for v in range(HEAD_DIM // 16)]
        for h in range(NUM_HEADS)
    ]

    def scale_repack(j):
        src = gbufs[j]
        dst = obufs[j]

        def rp_body(u, carry2):
            r = u * NUM_HEADS
            for h in range(NUM_HEADS):
                for v in range(HEAD_DIM // 16):
                    dst[u, 0, pl.ds(h * HEAD_DIM + v * 16, 16)] = (
                        src[r + h, pl.ds(v * 16, 16)] * gv[h][v]
                    )
            return carry2

        lax.fori_loop(0, CP * 8, rp_body, 0)

    fire_gather(0, 0)

    def pipe_body(t, carry):
        for jj in range(2):
            c = t * 2 + jj

            @pl.when(c + 1 < NCH)
            def _():
                fire_gather(c + 1, 1 - jj)

            wait_gather(jj)

            @pl.when(c >= 2)
            def _():
                wait_write(jj)

            scale_repack(jj)
            fire_write(c, jj)
        return carry

    lax.fori_loop(0, NCH // 2, pipe_body, 0)

    wait_write(0)
    wait_write(1)


@jax.jit
def kernel(curr, prev, table, gate):
    seqt = jnp.concatenate([prev, curr], axis=1).T
    seqt = jnp.pad(seqt, ((0, SEQ_PAD - SEQ_LEN), (0, 0)))
    table_flat = table.reshape(MEMORY_SIZE * NUM_HEADS, HEAD_DIM)
    return _build_engram_sc()(seqt, table_flat, gate)
